# baseline (speedup 1.0000x reference)
import functools
import math

import jax
import jax.numpy as jnp
import numpy as np
from jax import lax
from jax.experimental import pallas as pl
from jax.experimental.pallas import tpu as pltpu
from jax.experimental.pallas import tpu_sc as plsc

D_MODEL = 128
BATCH = 1024
T = 200
B_TOTAL = BATCH * T
NC, NS = 2, 16
NW = NC * NS
CHUNK = 128

N_SC = 27 * NW * CHUNK
N_TC = B_TOTAL - N_SC
TC_BLK = 1024
assert N_TC % TC_BLK == 0


def _make_sc_gather(n_rows):
    b_per_w = n_rows // NW
    n_chunks = b_per_w // CHUNK
    mesh = plsc.VectorSubcoreMesh(core_axis_name="c", subcore_axis_name="s")
    nbuf = 4

    @functools.partial(
        pl.kernel,
        mesh=mesh,
        out_type=jax.ShapeDtypeStruct((n_rows, D_MODEL), jnp.float32),
        scratch_types=[
            pltpu.VMEM((b_per_w,), jnp.int32),
        ]
        + [pltpu.VMEM((CHUNK, D_MODEL), jnp.float32)] * nbuf
        + [pltpu.SemaphoreType.DMA] * (2 * nbuf),
    )
    def gather_kernel(idx_hbm, table_hbm, out_hbm, idx_v, *scratch):
        bufs = scratch[:nbuf]
        gsems = scratch[nbuf : 2 * nbuf]
        ssems = scratch[2 * nbuf :]

        wid = lax.axis_index("s") * NC + lax.axis_index("c")
        base = wid * b_per_w
        pltpu.sync_copy(idx_hbm.at[pl.ds(base, b_per_w)], idx_v)

        def fire_g(c, b):
            pltpu.async_copy(
                table_hbm.at[idx_v.at[pl.ds(c * CHUNK, CHUNK)]], bufs[b], gsems[b]
            )

        def wait_g(b):
            pltpu.make_async_copy(
                table_hbm.at[pl.ds(0, CHUNK)], bufs[b], gsems[b]
            ).wait()

        def fire_s(c, b):
            pltpu.async_copy(
                bufs[b], out_hbm.at[pl.ds(base + c * CHUNK, CHUNK)], ssems[b]
            )

        def wait_s(b):
            pltpu.make_async_copy(
                bufs[b], out_hbm.at[pl.ds(base, CHUNK)], ssems[b]
            ).wait()

        fire_g(0, 0)
        fire_g(1, 1)
        for c in (0, 1):
            b = c % nbuf
            wait_g(b)
            fire_s(c, b)
            fire_g(c + 2, (c + 2) % nbuf)

        def ring(j, carry):
            for i in range(4):
                c = 2 + 4 * j + i
                b = (2 + i) % nbuf
                wait_g(b)
                fire_s(c, b)
                wait_s((b + 2) % nbuf)
                fire_g(c + 2, (b + 2) % nbuf)
            return carry

        n_uniform = n_chunks - 4
        lax.fori_loop(0, n_uniform // 4, ring, 0)
        for c in range(2 + (n_uniform // 4) * 4, n_chunks - 2):
            b = c % nbuf
            wait_g(b)
            fire_s(c, b)
            wait_s((b + 2) % nbuf)
            fire_g(c + 2, (b + 2) % nbuf)
        for c in (n_chunks - 2, n_chunks - 1):
            b = c % nbuf
            wait_g(b)
            fire_s(c, b)
            wait_s((b + 2) % nbuf)
        wait_s((n_chunks - 2) % nbuf)
        wait_s((n_chunks - 1) % nbuf)

    return gather_kernel


_sc_gather = _make_sc_gather(N_SC)


def _tc_body(idx_ref, div_ref, out_ref):
    d = idx_ref[...].astype(jnp.float32)
    x = d * div_ref[...]
    s = jnp.sin(x)
    c = jnp.cos(x)
    col = lax.broadcasted_iota(jnp.int32, (TC_BLK, D_MODEL), 1)
    out_ref[...] = jnp.where(col % 2 == 0, s, c)


_tc_encode = pl.pallas_call(
    _tc_body,
    out_shape=jax.ShapeDtypeStruct((N_TC, D_MODEL), jnp.float32),
    grid=(N_TC // TC_BLK,),
    in_specs=[
        pl.BlockSpec((TC_BLK, 1), lambda i: (i, 0)),
        pl.BlockSpec((1, D_MODEL), lambda i: (0, 0)),
    ],
    out_specs=pl.BlockSpec((TC_BLK, D_MODEL), lambda i: (i, 0)),
)

_DIV_FULL = jnp.asarray(
    np.repeat(
        np.exp(
            np.arange(0, D_MODEL, 2, dtype=np.float32)
            * -(math.log(10000.0) / D_MODEL)
        ),
        2,
    )
).reshape(1, D_MODEL)


def kernel(delta_t, pe):
    idx = delta_t.reshape(-1)
    table = pe[0]
    sc_out = _sc_gather(idx[:N_SC], table)
    tc_out = _tc_encode(idx[N_SC:].reshape(N_TC, 1), _DIV_FULL)
    out = jnp.concatenate([sc_out, tc_out], axis=0)
    return out.reshape(BATCH, T, D_MODEL)


# speedup vs baseline: 3.1360x; 3.1360x over previous
import functools

import jax
import jax.numpy as jnp
from jax import lax
from jax.experimental import pallas as pl
from jax.experimental.pallas import tpu as pltpu
from jax.experimental.pallas import tpu_sc as plsc

D_MODEL = 128
BATCH = 1024
T = 200
B_TOTAL = BATCH * T
NC, NS = 2, 16
NW = NC * NS
B_PER_W = B_TOTAL // NW
CHUNK = 128
N_CHUNKS = B_PER_W // CHUNK
NBUF = 6
DEPTH = NBUF // 2


def _make_gather():
    mesh = plsc.VectorSubcoreMesh(core_axis_name="c", subcore_axis_name="s")

    @functools.partial(
        pl.kernel,
        mesh=mesh,
        out_type=jax.ShapeDtypeStruct((B_TOTAL, D_MODEL), jnp.float32),
        scratch_types=[
            pltpu.VMEM((B_PER_W,), jnp.int32),
        ]
        + [pltpu.VMEM((CHUNK, D_MODEL), jnp.float32)] * NBUF
        + [pltpu.SemaphoreType.DMA] * (2 * NBUF),
    )
    def gather_kernel(idx_hbm, table_hbm, out_hbm, idx_v, *scratch):
        bufs = scratch[:NBUF]
        gsems = scratch[NBUF : 2 * NBUF]
        ssems = scratch[2 * NBUF :]

        wid = lax.axis_index("s") * NC + lax.axis_index("c")
        base = wid * B_PER_W
        pltpu.sync_copy(idx_hbm.at[pl.ds(base, B_PER_W)], idx_v)

        def fire_g(c, b):
            pltpu.async_copy(
                table_hbm.at[idx_v.at[pl.ds(c * CHUNK, CHUNK)]], bufs[b], gsems[b]
            )

        def wait_g(b):
            pltpu.make_async_copy(
                table_hbm.at[pl.ds(0, CHUNK)], bufs[b], gsems[b]
            ).wait()

        def fire_s(c, b):
            pltpu.async_copy(
                bufs[b], out_hbm.at[pl.ds(base + c * CHUNK, CHUNK)], ssems[b]
            )

        def wait_s(b):
            pltpu.make_async_copy(
                bufs[b], out_hbm.at[pl.ds(base, CHUNK)], ssems[b]
            ).wait()

        for c in range(DEPTH):
            fire_g(c, c % NBUF)
        for c in range(DEPTH):
            b = c % NBUF
            wait_g(b)
            fire_s(c, b)
            fire_g(c + DEPTH, (c + DEPTH) % NBUF)

        def steady(c, b):
            wait_g(b)
            fire_s(c, b)
            wait_s((b + DEPTH) % NBUF)
            fire_g(c + DEPTH, (b + DEPTH) % NBUF)

        n_mid = N_CHUNKS - 2 * DEPTH
        n_loop = n_mid // NBUF

        def ring(j, carry):
            for i in range(NBUF):
                steady(DEPTH + NBUF * j + i, (DEPTH + i) % NBUF)
            return carry

        lax.fori_loop(0, n_loop, ring, 0)
        for c in range(DEPTH + n_loop * NBUF, N_CHUNKS - DEPTH):
            steady(c, c % NBUF)
        for c in range(N_CHUNKS - DEPTH, N_CHUNKS):
            b = c % NBUF
            wait_g(b)
            fire_s(c, b)
            wait_s((b + DEPTH) % NBUF)
        for c in range(N_CHUNKS - DEPTH, N_CHUNKS):
            wait_s(c % NBUF)

    return gather_kernel


_gather = _make_gather()


def kernel(delta_t, pe):
    idx = delta_t.reshape(-1)
    table = pe[0]
    out = _gather(idx, table)
    return out.reshape(BATCH, T, D_MODEL)


# speedup vs baseline: 4.8379x; 1.5427x over previous
import functools

import jax
import jax.numpy as jnp
from jax import lax
from jax.experimental import pallas as pl
from jax.experimental.pallas import tpu as pltpu
from jax.experimental.pallas import tpu_sc as plsc

D_MODEL = 128
BATCH = 1024
T = 200
B_TOTAL = BATCH * T
NC, NS = 2, 16
NW = NC * NS
B_PER_W = B_TOTAL // NW
CHUNK = 128
N_CHUNKS = B_PER_W // CHUNK
NBUF = 4
DEPTH = NBUF // 2


def _make_gather():
    mesh = plsc.VectorSubcoreMesh(core_axis_name="c", subcore_axis_name="s")

    @functools.partial(
        pl.kernel,
        mesh=mesh,
        out_type=jax.ShapeDtypeStruct((B_TOTAL, D_MODEL), jnp.float32),
        scratch_types=[
            pltpu.VMEM((B_PER_W,), jnp.int32),
        ]
        + [pltpu.VMEM((CHUNK, D_MODEL), jnp.float32)] * NBUF
        + [pltpu.SemaphoreType.DMA] * (2 * NBUF)
        + [pltpu.VMEM_SHARED((5000, D_MODEL), jnp.float32)],
    )
    def gather_kernel(idx_hbm, table_hbm, out_hbm, idx_v, *scratch):
        bufs = scratch[:NBUF]
        gsems = scratch[NBUF : 2 * NBUF]
        ssems = scratch[2 * NBUF : 3 * NBUF]
        table_sp = scratch[3 * NBUF]

        sid = lax.axis_index("s")
        wid = sid * NC + lax.axis_index("c")
        base = wid * B_PER_W

        @pl.when(sid == 0)
        def _():
            pltpu.sync_copy(table_hbm, table_sp)

        plsc.subcore_barrier()

        pltpu.sync_copy(idx_hbm.at[pl.ds(base, B_PER_W)], idx_v)

        def fire_g(c, b):
            pltpu.async_copy(
                table_sp.at[idx_v.at[pl.ds(c * CHUNK, CHUNK)]], bufs[b], gsems[b]
            )

        def wait_g(b):
            pltpu.make_async_copy(
                table_hbm.at[pl.ds(0, CHUNK)], bufs[b], gsems[b]
            ).wait()

        def fire_s(c, b):
            pltpu.async_copy(
                bufs[b], out_hbm.at[pl.ds(base + c * CHUNK, CHUNK)], ssems[b]
            )

        def wait_s(b):
            pltpu.make_async_copy(
                bufs[b], out_hbm.at[pl.ds(base, CHUNK)], ssems[b]
            ).wait()

        for c in range(DEPTH):
            fire_g(c, c % NBUF)
        for c in range(DEPTH):
            b = c % NBUF
            wait_g(b)
            fire_s(c, b)
            fire_g(c + DEPTH, (c + DEPTH) % NBUF)

        def steady(c, b):
            wait_g(b)
            fire_s(c, b)
            wait_s((b + DEPTH) % NBUF)
            fire_g(c + DEPTH, (b + DEPTH) % NBUF)

        n_mid = N_CHUNKS - 2 * DEPTH
        n_loop = n_mid // NBUF

        def ring(j, carry):
            for i in range(NBUF):
                steady(DEPTH + NBUF * j + i, (DEPTH + i) % NBUF)
            return carry

        lax.fori_loop(0, n_loop, ring, 0)
        for c in range(DEPTH + n_loop * NBUF, N_CHUNKS - DEPTH):
            steady(c, c % NBUF)
        for c in range(N_CHUNKS - DEPTH, N_CHUNKS):
            b = c % NBUF
            wait_g(b)
            fire_s(c, b)
            wait_s((b + DEPTH) % NBUF)
        for c in range(N_CHUNKS - DEPTH, N_CHUNKS):
            wait_s(c % NBUF)

    return gather_kernel


_gather = _make_gather()


def kernel(delta_t, pe):
    idx = delta_t.reshape(-1)
    table = pe[0]
    out = _gather(idx, table)
    return out.reshape(BATCH, T, D_MODEL)
